# baseline (speedup 1.0000x reference)
import functools

import jax
import jax.numpy as jnp
from jax import lax
from jax.experimental import pallas as pl
from jax.experimental.pallas import tpu as pltpu
from jax.experimental.pallas import tpu_sc as plsc

B = 1024
L = 200
DIM = 32
V = 1000000
N = B * L

NC = 2
NS = 16
NW = NC * NS

BLK = 16384
G1 = (V + BLK - 1) // BLK
V4 = G1 * (BLK // 4)
PIECE = 256
K2_STEPS = (L + NW - 1) // NW


@functools.partial(
    pl.pallas_call,
    grid=(G1,),
    in_specs=[pl.BlockSpec((DIM, BLK), lambda i: (0, i))],
    out_specs=pl.BlockSpec((BLK // 4, 128), lambda i: (i, 0)),
    out_shape=jax.ShapeDtypeStruct((V4, 128), jnp.float32),
)
def _repack(x_ref, o_ref):
    x = x_ref[...].astype(jnp.bfloat16)
    o_ref[...] = jnp.concatenate(
        [
            jnp.concatenate(
                [
                    x[:, 2048 * p + 512 * a:2048 * p + 512 * a + 512].T
                    for a in range(4)
                ],
                axis=1,
            )
            for p in range(BLK // 2048)
        ],
        axis=0,
    ).astype(jnp.float32)


@functools.partial(
    pl.kernel,
    out_type=jax.ShapeDtypeStruct((L, PIECE, 4, DIM), jnp.float32),
    mesh=plsc.VectorSubcoreMesh(core_axis_name="c", subcore_axis_name="s"),
    scratch_types=[
        pltpu.VMEM((B,), jnp.int32),
        pltpu.VMEM((B,), jnp.int32),
        pltpu.VMEM((PIECE, DIM), jnp.float32),
        pltpu.VMEM((PIECE, DIM), jnp.float32),
        pltpu.SemaphoreType.DMA,
        pltpu.SemaphoreType.DMA,
    ],
    compiler_params=pltpu.CompilerParams(use_tc_tiling_on_sc=False),
)
def _gather(idsT_hbm, rm_hbm, out_hbm, ids_v, qm_v, buf0, buf1, sem0, sem1):
    bufs = (buf0, buf1)
    sems = (sem0, sem1)
    wid = lax.axis_index("s") * NC + lax.axis_index("c")

    def do_l(l):
        pltpu.sync_copy(idsT_hbm.at[l], ids_v)

        def qbody(g, _):
            v = ids_v[pl.ds(g * 16, 16)]
            hi = lax.bitwise_and(v, jnp.full((16,), -2048, jnp.int32))
            mid = lax.shift_left(
                lax.bitwise_and(v, jnp.full((16,), 511, jnp.int32)),
                jnp.full((16,), 2, jnp.int32),
            )
            lo = lax.bitwise_and(
                lax.shift_right_logical(v, jnp.full((16,), 9, jnp.int32)),
                jnp.full((16,), 3, jnp.int32),
            )
            qm_v[pl.ds(g * 16, 16)] = lax.bitwise_or(hi, lax.bitwise_or(mid, lo))
            return 0

        lax.fori_loop(0, B // 16, qbody, 0)

        cur = pltpu.async_copy(rm_hbm.at[qm_v.at[pl.ds(0, PIECE)]], buf0, sem0)
        for a in range(4):
            nxt = None
            if a < 3:
                nxt = pltpu.async_copy(
                    rm_hbm.at[qm_v.at[pl.ds((a + 1) * PIECE, PIECE)]],
                    bufs[(a + 1) % 2],
                    sems[(a + 1) % 2],
                )
            cur.wait()
            pltpu.sync_copy(bufs[a % 2], out_hbm.at[l, :, a, :])
            cur = nxt

    for k in range(K2_STEPS):
        l = wid + NW * k
        if k < K2_STEPS - 1:
            do_l(l)
        else:
            @pl.when(l < L)
            def _():
                do_l(l)


LPG = 8


@functools.partial(
    pl.pallas_call,
    grid=(L // LPG,),
    in_specs=[pl.BlockSpec((LPG * PIECE, 128), lambda i: (i, 0))],
    out_specs=pl.BlockSpec((LPG, DIM, B), lambda i: (i, 0, 0)),
    out_shape=jax.ShapeDtypeStruct((L, DIM, B), jnp.float32),
)
def _slabify(x_ref, o_ref):
    x = x_ref[...].astype(jnp.bfloat16)
    o_ref[...] = jnp.stack(
        [
            jnp.concatenate(
                [
                    x[PIECE * q:PIECE * (q + 1), 32 * a:32 * a + 32].T
                    for a in range(4)
                ],
                axis=1,
            )
            for q in range(LPG)
        ],
        axis=0,
    ).astype(jnp.float32)


def kernel(input_ids, attention_mask, table):
    rm = _repack(table.T)
    x2 = _gather(input_ids.T, rm.reshape(V4 * 4, DIM))
    outT = _slabify(x2.reshape(L * PIECE, 128))
    return jnp.transpose(outT, (2, 0, 1)), attention_mask


# speedup vs baseline: 1.0936x; 1.0936x over previous
import functools

import jax
import jax.numpy as jnp
from jax import lax
from jax.experimental import pallas as pl
from jax.experimental.pallas import tpu as pltpu
from jax.experimental.pallas import tpu_sc as plsc

B = 1024
L = 200
DIM = 32
V = 1000000
N = B * L

NC = 2
NS = 16
NW = NC * NS

BLK = 32768
G1 = (V + BLK - 1) // BLK
V4 = G1 * (BLK // 4)
PIECE = 256
K2_STEPS = (L + NW - 1) // NW


@functools.partial(
    pl.pallas_call,
    grid=(G1,),
    in_specs=[pl.BlockSpec((DIM, BLK), lambda i: (0, i))],
    out_specs=pl.BlockSpec((BLK // 4, 128), lambda i: (i, 0)),
    out_shape=jax.ShapeDtypeStruct((V4, 128), jnp.float32),
)
def _repack(x_ref, o_ref):
    x = x_ref[...].astype(jnp.bfloat16)
    o_ref[...] = jnp.concatenate(
        [
            jnp.concatenate(
                [
                    x[:, 2048 * p + 512 * a:2048 * p + 512 * a + 512].T
                    for a in range(4)
                ],
                axis=1,
            )
            for p in range(BLK // 2048)
        ],
        axis=0,
    ).astype(jnp.float32)


@functools.partial(
    pl.kernel,
    out_type=jax.ShapeDtypeStruct((L, PIECE, 4, DIM), jnp.float32),
    mesh=plsc.VectorSubcoreMesh(core_axis_name="c", subcore_axis_name="s"),
    scratch_types=[
        pltpu.VMEM((B,), jnp.int32),
        pltpu.VMEM((B,), jnp.int32),
        pltpu.VMEM((PIECE, DIM), jnp.float32),
        pltpu.VMEM((PIECE, DIM), jnp.float32),
        pltpu.SemaphoreType.DMA,
        pltpu.SemaphoreType.DMA,
    ],
    compiler_params=pltpu.CompilerParams(use_tc_tiling_on_sc=False),
)
def _gather(idsT_hbm, rm_hbm, out_hbm, ids_v, qm_v, buf0, buf1, sem0, sem1):
    bufs = (buf0, buf1)
    sems = (sem0, sem1)
    wid = lax.axis_index("s") * NC + lax.axis_index("c")

    def do_l(l):
        pltpu.sync_copy(idsT_hbm.at[l], ids_v)

        def qbody(g, _):
            v = ids_v[pl.ds(g * 16, 16)]
            hi = lax.bitwise_and(v, jnp.full((16,), -2048, jnp.int32))
            mid = lax.shift_left(
                lax.bitwise_and(v, jnp.full((16,), 511, jnp.int32)),
                jnp.full((16,), 2, jnp.int32),
            )
            lo = lax.bitwise_and(
                lax.shift_right_logical(v, jnp.full((16,), 9, jnp.int32)),
                jnp.full((16,), 3, jnp.int32),
            )
            qm_v[pl.ds(g * 16, 16)] = lax.bitwise_or(hi, lax.bitwise_or(mid, lo))
            return 0

        lax.fori_loop(0, B // 16, qbody, 0)

        cur = pltpu.async_copy(rm_hbm.at[qm_v.at[pl.ds(0, PIECE)]], buf0, sem0)
        for a in range(4):
            nxt = None
            if a < 3:
                nxt = pltpu.async_copy(
                    rm_hbm.at[qm_v.at[pl.ds((a + 1) * PIECE, PIECE)]],
                    bufs[(a + 1) % 2],
                    sems[(a + 1) % 2],
                )
            cur.wait()
            pltpu.sync_copy(bufs[a % 2], out_hbm.at[l, :, a, :])
            cur = nxt

    for k in range(K2_STEPS):
        l = wid + NW * k
        if k < K2_STEPS - 1:
            do_l(l)
        else:
            @pl.when(l < L)
            def _():
                do_l(l)


LPG = 8


@functools.partial(
    pl.pallas_call,
    grid=(L // LPG,),
    in_specs=[pl.BlockSpec((LPG * PIECE, 128), lambda i: (i, 0))],
    out_specs=pl.BlockSpec((LPG, DIM, B), lambda i: (i, 0, 0)),
    out_shape=jax.ShapeDtypeStruct((L, DIM, B), jnp.float32),
)
def _slabify(x_ref, o_ref):
    x = x_ref[...]
    o_ref[...] = jnp.stack(
        [
            jnp.concatenate(
                [
                    x[PIECE * q:PIECE * (q + 1), 32 * a:32 * a + 32].T
                    for a in range(4)
                ],
                axis=1,
            )
            for q in range(LPG)
        ],
        axis=0,
    )


def kernel(input_ids, attention_mask, table):
    rm = _repack(table.T)
    x2 = _gather(input_ids.T, rm.reshape(V4 * 4, DIM))
    outT = _slabify(x2.reshape(L * PIECE, 128))
    return jnp.transpose(outT, (2, 0, 1)), attention_mask
